# baseline (speedup 1.0000x reference)
import functools

import jax
import jax.numpy as jnp
from jax import lax
from jax.experimental import pallas as pl
from jax.experimental.pallas import tpu as pltpu
from jax.experimental.pallas import tpu_sc as plsc

_NC = 2
_NS = 16
_NW = _NC * _NS

_B = 4096 * 50
_D = 128
_BPW = _B // _NW
_CHUNK = 200
_NCHUNK = _BPW // _CHUNK
_LEAD = 2
_NBUF = 2 * _LEAD
_LOOP_END = _LEAD + _NBUF * ((_NCHUNK - 2 * _LEAD) // _NBUF)


@functools.partial(
    pl.kernel,
    out_type=jax.ShapeDtypeStruct((_B, _D), jnp.float32),
    mesh=plsc.VectorSubcoreMesh(
        core_axis_name="c", subcore_axis_name="s",
        num_cores=_NC, num_subcores=_NS),
    scratch_types=[
        pltpu.VMEM((_BPW,), jnp.int32),
        pltpu.VMEM((_NBUF, _CHUNK, _D), jnp.float32),
        pltpu.SemaphoreType.DMA,
        pltpu.SemaphoreType.DMA,
    ],
)
def _gather_kernel(table_hbm, idx_hbm, out_hbm, idx_v, rows_v, gsem, wsem):
    wid = lax.axis_index("s") * _NC + lax.axis_index("c")
    base = wid * _BPW
    pltpu.sync_copy(idx_hbm.at[pl.ds(base, _BPW)], idx_v)

    def g_desc(j, b):
        off = pl.multiple_of(j * _CHUNK, 8)
        return pltpu.make_async_copy(
            table_hbm.at[idx_v.at[pl.ds(off, _CHUNK)]], rows_v.at[b], gsem)

    def w_desc(j, b):
        off = pl.multiple_of(j * _CHUNK, 8)
        return pltpu.make_async_copy(
            rows_v.at[b], out_hbm.at[pl.ds(base + off, _CHUNK)], wsem)

    for jj in range(_LEAD):
        g_desc(jj, jj).start()
    for jj in range(_LEAD):
        g_desc(jj, jj).wait()
        w_desc(jj, jj).start()
        g_desc(jj + _LEAD, jj + _LEAD).start()

    @pl.loop(_LEAD, _LOOP_END, step=_NBUF)
    def _steady(j):
        for i in range(_NBUF):
            jj = j + i
            b = (_LEAD + i) % _NBUF
            bn = (b + _LEAD) % _NBUF
            g_desc(jj, b).wait()
            w_desc(jj, b).start()
            w_desc(jj - _LEAD, bn).wait()
            g_desc(jj + _LEAD, bn).start()

    for jj in range(_LOOP_END, _NCHUNK - _LEAD):
        b = jj % _NBUF
        bn = (b + _LEAD) % _NBUF
        g_desc(jj, b).wait()
        w_desc(jj, b).start()
        w_desc(jj - _LEAD, bn).wait()
        g_desc(jj + _LEAD, bn).start()
    for jj in range(_NCHUNK - _LEAD, _NCHUNK):
        b = jj % _NBUF
        g_desc(jj, b).wait()
        w_desc(jj, b).start()
        w_desc(jj - _LEAD, (b + _LEAD) % _NBUF).wait()
    for jj in range(_NCHUNK - _LEAD, _NCHUNK):
        w_desc(jj, jj % _NBUF).wait()


def kernel(class_embed_weight, indices):
    n_img, k = indices.shape
    idx_flat = jnp.transpose(indices).reshape(-1).astype(jnp.int32)
    out_flat = _gather_kernel(class_embed_weight, idx_flat)
    return out_flat.reshape(k, n_img, _D).transpose(1, 0, 2)


# speedup vs baseline: 1.0018x; 1.0018x over previous
import functools

import jax
import jax.numpy as jnp
from jax import lax
from jax.experimental import pallas as pl
from jax.experimental.pallas import tpu as pltpu
from jax.experimental.pallas import tpu_sc as plsc

_NC = 2
_NS = 16
_NW = _NC * _NS

_B = 4096 * 50
_D = 128
_BPW = _B // _NW
_CHUNK = 128
_NCHUNK = _BPW // _CHUNK
_LEAD = 3
_NBUF = 2 * _LEAD
_LOOP_END = _LEAD + _NBUF * ((_NCHUNK - 2 * _LEAD) // _NBUF)


@functools.partial(
    pl.kernel,
    out_type=jax.ShapeDtypeStruct((_B, _D), jnp.float32),
    mesh=plsc.VectorSubcoreMesh(
        core_axis_name="c", subcore_axis_name="s",
        num_cores=_NC, num_subcores=_NS),
    scratch_types=[
        pltpu.VMEM((_BPW,), jnp.int32),
        pltpu.VMEM((_NBUF, _CHUNK, _D), jnp.float32),
        pltpu.SemaphoreType.DMA,
        pltpu.SemaphoreType.DMA,
    ],
)
def _gather_kernel(table_hbm, idx_hbm, out_hbm, idx_v, rows_v, gsem, wsem):
    wid = lax.axis_index("s") * _NC + lax.axis_index("c")
    base = wid * _BPW
    pltpu.sync_copy(idx_hbm.at[pl.ds(base, _BPW)], idx_v)

    def g_desc(j, b):
        off = pl.multiple_of(j * _CHUNK, 8)
        return pltpu.make_async_copy(
            table_hbm.at[idx_v.at[pl.ds(off, _CHUNK)]], rows_v.at[b], gsem)

    def w_desc(j, b):
        off = pl.multiple_of(j * _CHUNK, 8)
        return pltpu.make_async_copy(
            rows_v.at[b], out_hbm.at[pl.ds(base + off, _CHUNK)], wsem)

    for jj in range(_LEAD):
        g_desc(jj, jj).start()
    for jj in range(_LEAD):
        g_desc(jj, jj).wait()
        w_desc(jj, jj).start()
        g_desc(jj + _LEAD, jj + _LEAD).start()

    @pl.loop(_LEAD, _LOOP_END, step=_NBUF)
    def _steady(j):
        for i in range(_NBUF):
            jj = j + i
            b = (_LEAD + i) % _NBUF
            bn = (b + _LEAD) % _NBUF
            g_desc(jj, b).wait()
            w_desc(jj, b).start()
            w_desc(jj - _LEAD, bn).wait()
            g_desc(jj + _LEAD, bn).start()

    for jj in range(_LOOP_END, _NCHUNK - _LEAD):
        b = jj % _NBUF
        bn = (b + _LEAD) % _NBUF
        g_desc(jj, b).wait()
        w_desc(jj, b).start()
        w_desc(jj - _LEAD, bn).wait()
        g_desc(jj + _LEAD, bn).start()
    for jj in range(_NCHUNK - _LEAD, _NCHUNK):
        b = jj % _NBUF
        g_desc(jj, b).wait()
        w_desc(jj, b).start()
        w_desc(jj - _LEAD, (b + _LEAD) % _NBUF).wait()
    for jj in range(_NCHUNK - _LEAD, _NCHUNK):
        w_desc(jj, jj % _NBUF).wait()


def kernel(class_embed_weight, indices):
    n_img, k = indices.shape
    idx_flat = jnp.transpose(indices).reshape(-1).astype(jnp.int32)
    out_flat = _gather_kernel(class_embed_weight, idx_flat)
    return out_flat.reshape(k, n_img, _D).transpose(1, 0, 2)
